# Initial kernel scaffold; baseline (speedup 1.0000x reference)
import jax
import jax.numpy as jnp
from jax.experimental import pallas as pl


def kernel(x, edge_index, W1, as1, ad1, b1, g1, be1, W2, as2, ad2, b2, g2, be2, W3, as3, ad3, b3):
    raise NotImplementedError("write your pallas kernel here")


# speedup vs baseline: 14.9113x; 14.9113x over previous
import functools

import jax
import jax.numpy as jnp
from jax import lax
from jax.experimental import pallas as pl
from jax.experimental.pallas import tpu as pltpu
from jax.experimental.pallas import tpu_sc as plsc

NC = 2
NS = 16
EB = 128

_BN_SCALE = 1.0 / (1.0 + 1e-5) ** 0.5


def _dense_tail(h, As_ref, Ad_ref, hlo_ref, hhi_ref, asrc_ref, adst_ref,
                m_ref, ms_s, md_s, i, last):
    fh = h.shape[1] // 2
    hlo_ref[...] = h[:, :fh]
    hhi_ref[...] = h[:, fh:]
    asrc = jnp.dot(h, As_ref[...], preferred_element_type=jnp.float32)
    adst = jnp.dot(h, Ad_ref[...], preferred_element_type=jnp.float32)
    asrc_ref[...] = asrc
    adst_ref[...] = adst
    ms = jnp.max(asrc, axis=0, keepdims=True)
    md = jnp.max(adst, axis=0, keepdims=True)

    @pl.when(i == 0)
    def _():
        ms_s[...] = ms
        md_s[...] = md

    @pl.when(i > 0)
    def _():
        ms_s[...] = jnp.maximum(ms_s[...], ms)
        md_s[...] = jnp.maximum(md_s[...], md)

    @pl.when(i == last)
    def _():
        t = ms_s[...] + md_s[...]
        m_ref[...] = jnp.where(t >= 0, t, 0.2 * t)


def _make_pre(n, rows, in_dim, f, heads):
    g = n // rows
    fh = f // 2

    def body(x_ref, W_ref, As_ref, Ad_ref, hlo_ref, hhi_ref, asrc_ref,
             adst_ref, m_ref, ms_s, md_s):
        i = pl.program_id(0)
        h = jnp.dot(x_ref[...], W_ref[...], preferred_element_type=jnp.float32)
        _dense_tail(h, As_ref, Ad_ref, hlo_ref, hhi_ref, asrc_ref, adst_ref,
                    m_ref, ms_s, md_s, i, g - 1)

    full = lambda shape: pl.BlockSpec(shape, lambda i: (0, 0))
    rb = lambda shape: pl.BlockSpec(shape, lambda i: (i, 0))
    return pl.pallas_call(
        body,
        grid=(g,),
        in_specs=[rb((rows, in_dim)), full((in_dim, f)), full((f, 16)),
                  full((f, 16))],
        out_specs=[rb((rows, fh)), rb((rows, fh)), rb((rows, 16)),
                   rb((rows, 16)), full((1, 16))],
        out_shape=[
            jax.ShapeDtypeStruct((n, fh), jnp.float32),
            jax.ShapeDtypeStruct((n, fh), jnp.float32),
            jax.ShapeDtypeStruct((n, 16), jnp.float32),
            jax.ShapeDtypeStruct((n, 16), jnp.float32),
            jax.ShapeDtypeStruct((1, 16), jnp.float32),
        ],
        scratch_shapes=[pltpu.VMEM((1, 16), jnp.float32),
                        pltpu.VMEM((1, 16), jnp.float32)],
    )


def _make_mid(n, rows, hp, cp, f_next, heads_next, final_only=False):
    g = n // rows
    fp = hp * cp

    def norm(lo_ref, hi_ref, den_ref, b_ref):
        acc = jnp.concatenate([lo_ref[...], hi_ref[...]], axis=1)
        parts = [acc[:, h * cp:(h + 1) * cp] / (den_ref[:, h:h + 1] + 1e-16)
                 for h in range(hp)]
        return jnp.concatenate(parts, axis=1) + b_ref[...]

    if final_only:
        def body(lo_ref, hi_ref, den_ref, b_ref, out_ref):
            y = norm(lo_ref, hi_ref, den_ref, b_ref)
            y = y - jnp.max(y, axis=1, keepdims=True)
            out_ref[...] = y - jnp.log(
                jnp.sum(jnp.exp(y), axis=1, keepdims=True))

        full = lambda shape: pl.BlockSpec(shape, lambda i: (0, 0))
        rb = lambda shape: pl.BlockSpec(shape, lambda i: (i, 0))
        return pl.pallas_call(
            body,
            grid=(g,),
            in_specs=[rb((rows, fp // 2)), rb((rows, fp // 2)),
                      rb((rows, 16)), full((1, fp))],
            out_specs=[rb((rows, fp))],
            out_shape=[jax.ShapeDtypeStruct((n, fp), jnp.float32)],
        )

    fh = f_next // 2

    def body(lo_ref, hi_ref, den_ref, b_ref, g_ref, be_ref, W_ref, As_ref,
             Ad_ref, hlo_ref, hhi_ref, asrc_ref, adst_ref, m_ref, ms_s, md_s):
        i = pl.program_id(0)
        y = norm(lo_ref, hi_ref, den_ref, b_ref)
        y = y * _BN_SCALE * g_ref[...] + be_ref[...]
        hcur = jnp.where(y > 0, y, jnp.exp(y) - 1.0)
        h = jnp.dot(hcur, W_ref[...], preferred_element_type=jnp.float32)
        _dense_tail(h, As_ref, Ad_ref, hlo_ref, hhi_ref, asrc_ref, adst_ref,
                    m_ref, ms_s, md_s, i, g - 1)

    full = lambda shape: pl.BlockSpec(shape, lambda i: (0, 0))
    rb = lambda shape: pl.BlockSpec(shape, lambda i: (i, 0))
    return pl.pallas_call(
        body,
        grid=(g,),
        in_specs=[rb((rows, fp // 2)), rb((rows, fp // 2)), rb((rows, 16)),
                  full((1, fp)), full((1, fp)), full((1, fp)),
                  full((fp, f_next)), full((f_next, 16)),
                  full((f_next, 16))],
        out_specs=[rb((rows, fh)), rb((rows, fh)), rb((rows, 16)),
                   rb((rows, 16)), full((1, 16))],
        out_shape=[
            jax.ShapeDtypeStruct((n, fh), jnp.float32),
            jax.ShapeDtypeStruct((n, fh), jnp.float32),
            jax.ShapeDtypeStruct((n, 16), jnp.float32),
            jax.ShapeDtypeStruct((n, 16), jnp.float32),
            jax.ShapeDtypeStruct((1, 16), jnp.float32),
        ],
        scratch_shapes=[pltpu.VMEM((1, 16), jnp.float32),
                        pltpu.VMEM((1, 16), jnp.float32)],
    )


def _make_edge(n, e, f, heads):
    fh = f // 2
    nblk = e // EB
    base_blk, rem_blk = nblk // NS, nblk % NS
    rows_a = (n // NS) // 8 * 8
    rows_b = n - (NS - 1) * rows_a
    chunks = fh // 16
    heads_local = max(1, heads // 2)
    cpj = chunks // heads_local

    mesh = plsc.VectorSubcoreMesh(core_axis_name="c", subcore_axis_name="s",
                                  num_cores=NC, num_subcores=NS)

    @functools.partial(
        pl.kernel,
        mesh=mesh,
        compiler_params=pltpu.CompilerParams(use_tc_tiling_on_sc=False),
        out_type=[
            jax.ShapeDtypeStruct((n, fh), jnp.float32),
            jax.ShapeDtypeStruct((n, fh), jnp.float32),
            jax.ShapeDtypeStruct((n, 16), jnp.float32),
        ],
        scratch_types=[
            pltpu.VMEM((1, 16), jnp.float32),
            pltpu.VMEM((EB,), jnp.int32),
            pltpu.VMEM((EB,), jnp.int32),
            pltpu.VMEM((EB, 16), jnp.float32),
            pltpu.VMEM((EB, 16), jnp.float32),
            pltpu.VMEM((EB, 16), jnp.float32),
            pltpu.VMEM((EB, fh), jnp.float32),
            pltpu.VMEM_SHARED((n, fh), jnp.float32),
            pltpu.VMEM_SHARED((n, 16), jnp.float32),
            pltpu.SemaphoreType.DMA,
        ],
    )
    def k(hlo_hbm, hhi_hbm, asrc_hbm, adst_hbm, m_hbm, src_hbm, dst_hbm,
          zf_hbm, zh_hbm, out_lo, out_hi, den_out,
          m_v, src_v, dst_v, sg_v, dg_v, w_v, rows_v, acc_s, dacc_s, sem):
        c = lax.axis_index("c")
        s = lax.axis_index("s")

        def stripe_copy(fn):
            @pl.when(s < NS - 1)
            def _():
                fn(pl.ds(s * rows_a, rows_a))

            @pl.when(s == NS - 1)
            def _():
                fn(pl.ds((NS - 1) * rows_a, rows_b))

        pltpu.sync_copy(m_hbm, m_v)
        stripe_copy(lambda st: pltpu.sync_copy(zf_hbm.at[st], acc_s.at[st]))

        @pl.when(c == 0)
        def _():
            stripe_copy(
                lambda st: pltpu.sync_copy(zh_hbm.at[st], dacc_s.at[st]))

        plsc.subcore_barrier()

        mvec = m_v[0, :]

        def edge_loop(h_hbm, head_base, do_den):
            def blk_body(i, _):
                eoff = (i * NS + s) * EB
                pltpu.sync_copy(src_hbm.at[pl.ds(eoff, EB)], src_v)
                pltpu.sync_copy(dst_hbm.at[pl.ds(eoff, EB)], dst_v)
                pltpu.async_copy(asrc_hbm.at[src_v], sg_v, sem).wait()
                pltpu.async_copy(adst_hbm.at[dst_v], dg_v, sem).wait()
                pltpu.async_copy(h_hbm.at[src_v], rows_v, sem).wait()

                def edge(ei, _):
                    av = sg_v[ei, :] + dg_v[ei, :]
                    l = jnp.where(av >= 0, av, 0.2 * av)
                    w16 = jnp.exp(l - mvec)
                    if do_den:
                        w_v[ei, :] = w16
                    for hl in range(heads_local):
                        wv = w16[head_base + hl]
                        for kc in range(cpj):
                            sl = pl.ds((hl * cpj + kc) * 16, 16)
                            rows_v[ei, sl] = rows_v[ei, sl] * wv
                    return 0

                lax.fori_loop(0, EB, edge, 0)
                pltpu.sync_copy(rows_v, acc_s.at[dst_v], add=True)
                if do_den:
                    pltpu.sync_copy(w_v, dacc_s.at[dst_v], add=True)
                return 0

            lax.fori_loop(0, base_blk, blk_body, 0)

            @pl.when(s < rem_blk)
            def _():
                blk_body(base_blk, 0)

        @pl.when(c == 0)
        def _():
            edge_loop(hlo_hbm, 0, True)

        @pl.when(c == 1)
        def _():
            edge_loop(hhi_hbm, heads - heads_local if heads > 1 else 0,
                      False)

        plsc.subcore_barrier()

        @pl.when(c == 0)
        def _():
            stripe_copy(lambda st: pltpu.sync_copy(acc_s.at[st],
                                                   out_lo.at[st]))
            stripe_copy(lambda st: pltpu.sync_copy(dacc_s.at[st],
                                                   den_out.at[st]))

        @pl.when(c == 1)
        def _():
            stripe_copy(lambda st: pltpu.sync_copy(acc_s.at[st],
                                                   out_hi.at[st]))

    return k


def _att_mat(a):
    h, c = a.shape
    m = jnp.zeros((h, c, 16), jnp.float32)
    m = m.at[jnp.arange(h), :, jnp.arange(h)].set(a)
    return m.reshape(h * c, 16)


def kernel(x, edge_index, W1, as1, ad1, b1, g1, be1, W2, as2, ad2, b2, g2,
           be2, W3, as3, ad3, b3):
    n, in_dim = x.shape
    e = edge_index.shape[1]
    f1 = W1.shape[1]
    h1_heads = as1.shape[0]
    f2 = W2.shape[1]
    f3 = W3.shape[1]
    rows = 1000

    src = edge_index[0]
    dst = edge_index[1]
    As1, Ad1 = _att_mat(as1), _att_mat(ad1)
    As2, Ad2 = _att_mat(as2), _att_mat(ad2)
    As3, Ad3 = _att_mat(as3), _att_mat(ad3)

    hlo, hhi, a_s, a_d, m = _make_pre(n, rows, in_dim, f1, h1_heads)(
        x, W1, As1, Ad1)
    zf1 = jnp.zeros((n, f1 // 2), jnp.float32)
    zh16 = jnp.zeros((n, 16), jnp.float32)
    acc_lo, acc_hi, den = _make_edge(n, e, f1, h1_heads)(
        hlo, hhi, a_s, a_d, m, src, dst, zf1, zh16)

    hlo, hhi, a_s, a_d, m = _make_mid(n, rows, h1_heads, f1 // h1_heads,
                                      f2, 1)(
        acc_lo, acc_hi, den, b1.reshape(1, -1), g1.reshape(1, -1),
        be1.reshape(1, -1), W2, As2, Ad2)
    zf2 = jnp.zeros((n, f2 // 2), jnp.float32)
    acc_lo, acc_hi, den = _make_edge(n, e, f2, 1)(
        hlo, hhi, a_s, a_d, m, src, dst, zf2, zh16)

    hlo, hhi, a_s, a_d, m = _make_mid(n, rows, 1, f2, f3, 1)(
        acc_lo, acc_hi, den, b2.reshape(1, -1), g2.reshape(1, -1),
        be2.reshape(1, -1), W3, As3, Ad3)
    zf3 = jnp.zeros((n, f3 // 2), jnp.float32)
    acc_lo, acc_hi, den = _make_edge(n, e, f3, 1)(
        hlo, hhi, a_s, a_d, m, src, dst, zf3, zh16)

    (out,) = _make_mid(n, rows, 1, f3, 0, 0, final_only=True)(
        acc_lo, acc_hi, den, b3.reshape(1, -1))
    return out
